# baseline (speedup 1.0000x reference)
import jax
import jax.numpy as jnp
from jax.experimental import pallas as pl
from jax.experimental.pallas import tpu as pltpu

_N = 10000
_DIN = 256
_DOUT = 256

_CM = 200
_NCH = _N // _CM
_DEPTH = 3


def _gcn_body(adj_hbm, xf_ref, w_ref, out_ref, buf, s_ref, sems):
    def chunk_copy(c, slot):
        return pltpu.make_async_copy(
            adj_hbm.at[pl.ds(c * _CM, _CM), :], buf.at[slot], sems.at[slot])

    for p in range(_DEPTH):
        chunk_copy(p, p).start()

    s_ref[...] = jnp.dot(xf_ref[...], w_ref[...],
                         preferred_element_type=jnp.float32
                         ).astype(jnp.bfloat16)

    for c in range(_NCH):
        slot = c % _DEPTH
        chunk_copy(c, slot).wait()
        out_ref[pl.ds(c * _CM, _CM), :] = jnp.dot(
            buf[slot].astype(jnp.bfloat16), s_ref[...],
            preferred_element_type=jnp.float32)
        if c + _DEPTH < _NCH:
            chunk_copy(c + _DEPTH, slot).start()


@jax.jit
def kernel(x, adj, W):
    xf = x.reshape(_N, _DIN)
    out = pl.pallas_call(
        _gcn_body,
        in_specs=[
            pl.BlockSpec(memory_space=pl.ANY),
            pl.BlockSpec((_N, _DIN), lambda: (0, 0)),
            pl.BlockSpec((_DIN, _DOUT), lambda: (0, 0)),
        ],
        out_specs=pl.BlockSpec((_N, _DOUT), lambda: (0, 0)),
        out_shape=jax.ShapeDtypeStruct((_N, _DOUT), jnp.float32),
        scratch_shapes=[
            pltpu.VMEM((_DEPTH, _CM, _N), jnp.float32),
            pltpu.VMEM((_N, _DOUT), jnp.bfloat16),
            pltpu.SemaphoreType.DMA((_DEPTH,)),
        ],
    )(adj, xf, W)
    return out
